# baseline (speedup 1.0000x reference)
import functools

import jax
import jax.numpy as jnp
from jax import lax
from jax.experimental import pallas as pl
from jax.experimental.pallas import tpu as pltpu
from jax.experimental.pallas import tpu_sc as plsc

_N, _D, _K = 10000, 128, 32
_RB = 2000
_S = _N // _RB
_KP = _K + 1
_TOTAL = _N * _KP
_NC, _NS = 2, 16
_NW = _NC * _NS
_R = 80
_L = 16
_NCH = _N // _R
_NQ = _KP * _NCH
_NB = 8
_CPW = -(-_NQ // _NW)
_QPW = -(-_CPW // _NB) * _NB
_BPW = _CPW * _R
_PAD = _NW * _BPW - _K * _N


def _prep_body(pf_ref, acc_ref, out_ref):
    phase = pl.program_id(0)

    @pl.when(jnp.logical_and(phase == 0, pl.program_id(1) == 0))
    def _init():
        acc_ref[...] = jnp.zeros_like(acc_ref)

    pf = pf_ref[...]

    @pl.when(phase == 0)
    def _accum():
        part = jnp.sum(pf * pf, axis=0, keepdims=True)
        acc_ref[...] += jnp.broadcast_to(part, acc_ref.shape)

    ss = acc_ref[0:1, :]
    norm = jnp.maximum(jnp.sqrt(ss), 1e-12)
    col = lax.broadcasted_iota(jnp.int32, (1, _D), 1)
    rgb = (col >= 3) & (col < 6)
    scale = jnp.where(rgb, 1.0 / 255.0, 1.0 / norm)
    out_ref[...] = pf * scale


def _prep(pf):
    return pl.pallas_call(
        _prep_body,
        grid=(2, _S),
        in_specs=[pl.BlockSpec((_RB, _D), lambda p, i: (i, 0))],
        out_specs=[
            pl.BlockSpec((8, _D), lambda p, i: (0, 0)),
            pl.BlockSpec((_RB, _D), lambda p, i: (i, 0)),
        ],
        out_shape=[
            jax.ShapeDtypeStruct((8, _D), jnp.float32),
            jax.ShapeDtypeStruct((_N, _D), jnp.float32),
        ],
    )(pf)[1]


@functools.cache
def _sc_gather_fn():
    mesh = plsc.VectorSubcoreMesh(core_axis_name="c", subcore_axis_name="s")

    @functools.partial(
        pl.kernel,
        mesh=mesh,
        out_type=jax.ShapeDtypeStruct((_N, _KP * _D), jnp.float32),
        scratch_types=[
            pltpu.VMEM((_BPW,), jnp.int32),
            *[pltpu.VMEM((_R, _D), jnp.float32) for _ in range(_NB)],
            *[pltpu.SemaphoreType.DMA for _ in range(2 * _NB)],
        ],
    )
    def _sc_gather(table_hbm, idx_hbm, out_hbm, idx_v, *bufs_sems):
        bufs = bufs_sems[:_NB]
        gsems = bufs_sems[_NB:2 * _NB]
        wsems = bufs_sems[2 * _NB:]
        wid = lax.axis_index("s") * _NC + lax.axis_index("c")
        base = wid * _BPW
        pltpu.sync_copy(idx_hbm.at[pl.ds(base, _BPW)], idx_v)

        def _split(q):
            ct = q // _NCH
            r0 = (q - ct * _NCH) * _R
            return ct, r0

        def _dst(q):
            ct, r0 = _split(q)
            return out_hbm.at[pl.ds(r0, _R), pl.ds(ct * _D, _D)]

        @pl.loop(0, _QPW // _NB)
        def _block(i):
            for b in range(_NB):
                k = i * _NB + b
                q = wid * _CPW + k

                @pl.when(jnp.logical_and(i > 0, jnp.logical_and(k < _CPW, q < _NQ)))
                def _wait_write():
                    pltpu.make_async_copy(bufs[b], _dst(q), wsems[b]).wait()

                @pl.when(jnp.logical_and(k < _CPW, q < _NQ))
                def _fix_and_gather():
                    _, r0 = _split(q)
                    for j in range(_R // _L):
                        off = k * _R + j * _L
                        v = idx_v[pl.ds(off, _L)]
                        rowv = r0 + j * _L + lax.iota(jnp.int32, _L)
                        idx_v[pl.ds(off, _L)] = jnp.where(v == 0, rowv, v)
                    pltpu.async_copy(
                        table_hbm.at[idx_v.at[pl.ds(k * _R, _R)]],
                        bufs[b], gsems[b],
                    )

            for b in range(_NB):
                k = i * _NB + b
                q = wid * _CPW + k

                @pl.when(jnp.logical_and(k < _CPW, q < _NQ))
                def _write():
                    pltpu.make_async_copy(
                        table_hbm.at[idx_v.at[pl.ds(k * _R, _R)]],
                        bufs[b], gsems[b],
                    ).wait()
                    pltpu.async_copy(bufs[b], _dst(q), wsems[b])

        for b in range(_NB):
            k = (_QPW // _NB - 1) * _NB + b
            q = wid * _CPW + k

            @pl.when(jnp.logical_and(k < _CPW, q < _NQ))
            def _drain():
                pltpu.make_async_copy(bufs[b], _dst(q), wsems[b]).wait()

    return _sc_gather


def kernel(points_features, points_neighbor):
    pf_n = _prep(points_features)
    idx_flat = jnp.pad(points_neighbor.T.reshape(-1), (0, _PAD))
    return _sc_gather_fn()(pf_n, idx_flat)


# speedup vs baseline: 1.0378x; 1.0378x over previous
import functools

import jax
import jax.numpy as jnp
from jax import lax
from jax.experimental import pallas as pl
from jax.experimental.pallas import tpu as pltpu
from jax.experimental.pallas import tpu_sc as plsc

_N, _D, _K = 10000, 128, 32
_RB = 10000
_S = _N // _RB
_KP = _K + 1
_TOTAL = _N * _KP
_NC, _NS = 2, 16
_NW = _NC * _NS
_R = 80
_L = 16
_NCH = _N // _R
_NQ = _KP * _NCH
_NB = 8
_CPW = -(-_NQ // _NW)
_QPW = -(-_CPW // _NB) * _NB
_BPW = _CPW * _R
_PAD = _NW * _BPW - _K * _N


def _prep_body(pf_ref, acc_ref, out_ref):
    phase = pl.program_id(0)

    @pl.when(jnp.logical_and(phase == 0, pl.program_id(1) == 0))
    def _init():
        acc_ref[...] = jnp.zeros_like(acc_ref)

    pf = pf_ref[...]

    @pl.when(phase == 0)
    def _accum():
        part = jnp.sum(pf * pf, axis=0, keepdims=True)
        acc_ref[...] += jnp.broadcast_to(part, acc_ref.shape)

    ss = acc_ref[0:1, :]
    norm = jnp.maximum(jnp.sqrt(ss), 1e-12)
    col = lax.broadcasted_iota(jnp.int32, (1, _D), 1)
    rgb = (col >= 3) & (col < 6)
    scale = jnp.where(rgb, 1.0 / 255.0, 1.0 / norm)
    out_ref[...] = pf * scale


def _prep(pf):
    return pl.pallas_call(
        _prep_body,
        grid=(2, _S),
        in_specs=[pl.BlockSpec((_RB, _D), lambda p, i: (i, 0))],
        out_specs=[
            pl.BlockSpec((8, _D), lambda p, i: (0, 0)),
            pl.BlockSpec((_RB, _D), lambda p, i: (i, 0)),
        ],
        out_shape=[
            jax.ShapeDtypeStruct((8, _D), jnp.float32),
            jax.ShapeDtypeStruct((_N, _D), jnp.float32),
        ],
    )(pf)[1]


@functools.cache
def _sc_gather_fn():
    mesh = plsc.VectorSubcoreMesh(core_axis_name="c", subcore_axis_name="s")

    @functools.partial(
        pl.kernel,
        mesh=mesh,
        out_type=jax.ShapeDtypeStruct((_N, _KP * _D), jnp.float32),
        scratch_types=[
            pltpu.VMEM((_BPW,), jnp.int32),
            *[pltpu.VMEM((_R, _D), jnp.float32) for _ in range(_NB)],
            *[pltpu.SemaphoreType.DMA for _ in range(2 * _NB)],
        ],
    )
    def _sc_gather(table_hbm, idx_hbm, out_hbm, idx_v, *bufs_sems):
        bufs = bufs_sems[:_NB]
        gsems = bufs_sems[_NB:2 * _NB]
        wsems = bufs_sems[2 * _NB:]
        wid = lax.axis_index("s") * _NC + lax.axis_index("c")
        base = wid * _BPW
        pltpu.sync_copy(idx_hbm.at[pl.ds(base, _BPW)], idx_v)

        def _split(q):
            ct = q // _NCH
            r0 = (q - ct * _NCH) * _R
            return ct, r0

        def _dst(q):
            ct, r0 = _split(q)
            return out_hbm.at[pl.ds(r0, _R), pl.ds(ct * _D, _D)]

        @pl.loop(0, _QPW // _NB)
        def _block(i):
            for b in range(_NB):
                k = i * _NB + b
                q = wid * _CPW + k

                @pl.when(jnp.logical_and(i > 0, jnp.logical_and(k < _CPW, q < _NQ)))
                def _wait_write():
                    pltpu.make_async_copy(bufs[b], _dst(q), wsems[b]).wait()

                @pl.when(jnp.logical_and(k < _CPW, q < _NQ))
                def _fix_and_gather():
                    _, r0 = _split(q)
                    for j in range(_R // _L):
                        off = k * _R + j * _L
                        v = idx_v[pl.ds(off, _L)]
                        rowv = r0 + j * _L + lax.iota(jnp.int32, _L)
                        idx_v[pl.ds(off, _L)] = jnp.where(v == 0, rowv, v)
                    pltpu.async_copy(
                        table_hbm.at[idx_v.at[pl.ds(k * _R, _R)]],
                        bufs[b], gsems[b],
                    )

            for b in range(_NB):
                k = i * _NB + b
                q = wid * _CPW + k

                @pl.when(jnp.logical_and(k < _CPW, q < _NQ))
                def _write():
                    pltpu.make_async_copy(
                        table_hbm.at[idx_v.at[pl.ds(k * _R, _R)]],
                        bufs[b], gsems[b],
                    ).wait()
                    pltpu.async_copy(bufs[b], _dst(q), wsems[b])

        for b in range(_NB):
            k = (_QPW // _NB - 1) * _NB + b
            q = wid * _CPW + k

            @pl.when(jnp.logical_and(k < _CPW, q < _NQ))
            def _drain():
                pltpu.make_async_copy(bufs[b], _dst(q), wsems[b]).wait()

    return _sc_gather


def kernel(points_features, points_neighbor):
    pf_n = _prep(points_features)
    idx_flat = jnp.pad(points_neighbor.T.reshape(-1), (0, _PAD))
    return _sc_gather_fn()(pf_n, idx_flat)
